# Initial kernel scaffold; baseline (speedup 1.0000x reference)
import jax
import jax.numpy as jnp
from jax.experimental import pallas as pl


def kernel(text, visual, audio, bank, W_t, b_t, W_v, b_v, W_a, b_a, prototypes):
    raise NotImplementedError("write your pallas kernel here")


# speedup vs baseline: 1.6627x; 1.6627x over previous
import functools

import jax
import jax.numpy as jnp
from jax import lax
from jax.experimental import pallas as pl
from jax.experimental.pallas import tpu as pltpu
from jax.experimental.pallas import tpu_sc as plsc

KNN = 10
TEMP = 0.1
TILE_K = 2048
BIG = 2 ** 30


def _proj_kernel(t_ref, v_ref, a_ref, wt_ref, bt_ref, wv_ref, bv_ref,
                 wa_ref, ba_ref, fused_ref, qn_ref):
    def proj(x, w, b):
        y = lax.dot_general(x, w, (((1,), (0,)), ((), ())),
                            preferred_element_type=jnp.float32) + b
        m = jnp.mean(y, axis=1, keepdims=True)
        yc = y - m
        var = jnp.mean(yc * yc, axis=1, keepdims=True)
        return jnp.maximum(yc / jnp.sqrt(var + 1e-5), 0.0)

    t = proj(t_ref[:, :], wt_ref[:, :], bt_ref[:, :])
    v = proj(v_ref[:, :], wv_ref[:, :], bv_ref[:, :])
    a = proj(a_ref[:, :], wa_ref[:, :], ba_ref[:, :])
    fused = (t + v + a) / 3.0
    fused_ref[:, :] = fused
    nrm = jnp.sqrt(jnp.sum(fused * fused, axis=1, keepdims=True))
    qn_ref[:, :] = fused / (nrm + 1e-12)


def _sims_topk_kernel(qn_ref, bank_ref, vals_ref, idx_ref, *, n_bank, b_rows):
    j = pl.program_id(0)

    @pl.when(j == 0)
    def _init():
        vals_ref[:, :] = jnp.full((b_rows, KNN), -4.0, jnp.float32)
        idx_ref[:, :] = jnp.zeros((b_rows, KNN), jnp.int32)

    bank = bank_ref[:, :]
    nrm = jnp.sqrt(jnp.sum(bank * bank, axis=1, keepdims=True))
    bank_n = bank / (nrm + 1e-12)
    sims = lax.dot_general(qn_ref[:, :], bank_n, (((1,), (1,)), ((), ())),
                           preferred_element_type=jnp.float32)
    col = jax.lax.broadcasted_iota(jnp.int32, sims.shape, 1) + j * TILE_K
    sims = jnp.where(col < n_bank, sims, -2.0)

    tvs, tis = [], []
    x = sims
    for _ in range(KNN):
        m = jnp.max(x, axis=1)
        eq = x == m[:, None]
        it = jnp.min(jnp.where(eq, col, BIG), axis=1)
        x = jnp.where(col == it[:, None], -3.0, x)
        tvs.append(m)
        tis.append(it)

    cv = jnp.concatenate([vals_ref[:, :], jnp.stack(tvs, axis=1)], axis=1)
    ci = jnp.concatenate([idx_ref[:, :], jnp.stack(tis, axis=1)], axis=1)
    posi = jax.lax.broadcasted_iota(jnp.int32, cv.shape, 1)
    nvs, nis = [], []
    for _ in range(KNN):
        m = jnp.max(cv, axis=1)
        eq = cv == m[:, None]
        pos = jnp.min(jnp.where(eq, posi, BIG), axis=1)
        sel = posi == pos[:, None]
        nis.append(jnp.sum(jnp.where(sel, ci, 0), axis=1))
        cv = jnp.where(sel, -5.0, cv)
        nvs.append(m)
    vals_ref[:, :] = jnp.stack(nvs, axis=1)
    idx_ref[:, :] = jnp.stack(nis, axis=1)


def _finish_kernel(fused_ref, vals_ref, nb_ref, proto_ref, logits_ref,
                   *, b_rows):
    z = vals_ref[:, :] / TEMP
    z = z - jnp.max(z, axis=1, keepdims=True)
    e = jnp.exp(z)
    w = e / jnp.sum(e, axis=1, keepdims=True)
    retrieved = jnp.zeros((b_rows, nb_ref.shape[2]), jnp.float32)
    for k in range(KNN):
        retrieved = retrieved + w[:, k:k + 1] * nb_ref[:, k, :]
    out = fused_ref[:, :] + retrieved
    onrm = jnp.sqrt(jnp.sum(out * out, axis=1, keepdims=True))
    on = out / (onrm + 1e-12)
    p = proto_ref[:, :]
    pnrm = jnp.sqrt(jnp.sum(p * p, axis=1, keepdims=True))
    pn = p / (pnrm + 1e-12)
    logits_ref[:, :] = lax.dot_general(
        on, pn, (((1,), (1,)), ((), ())),
        preferred_element_type=jnp.float32) / TEMP


def _sc_gather(bank, flat_idx, n_rows, dim):
    info = plsc.get_sparse_core_info()
    nw = info.num_cores * info.num_subcores
    b_per_w = n_rows // nw
    ch = None
    for c in (128, 120, 80, 64, 40, 32, 16, 8):
        if b_per_w % c == 0:
            ch = c
            break
    n_ch = b_per_w // ch
    mesh = plsc.VectorSubcoreMesh(core_axis_name="c", subcore_axis_name="s")

    @functools.partial(
        pl.kernel, mesh=mesh,
        out_type=jax.ShapeDtypeStruct((n_rows, dim), jnp.float32),
        scratch_types=[
            pltpu.VMEM((ch,), jnp.int32),
            pltpu.VMEM((ch, dim), jnp.float32),
            pltpu.SemaphoreType.DMA,
        ],
    )
    def gather_k(bank_hbm, idx_hbm, out_hbm, idx_v, rows_v, sem):
        wid = lax.axis_index("s") * info.num_cores + lax.axis_index("c")
        base = wid * b_per_w
        for c in range(n_ch):
            off = base + c * ch
            pltpu.sync_copy(idx_hbm.at[pl.ds(off, ch)], idx_v)
            pltpu.async_copy(bank_hbm.at[idx_v], rows_v, sem).wait()
            pltpu.sync_copy(rows_v, out_hbm.at[pl.ds(off, ch)])

    return gather_k(bank, flat_idx)


def kernel(text, visual, audio, bank, W_t, b_t, W_v, b_v, W_a, b_a, prototypes):
    B = text.shape[0]
    K, D = bank.shape
    C = prototypes.shape[0]

    fused, qn = pl.pallas_call(
        _proj_kernel,
        out_shape=[jax.ShapeDtypeStruct((B, D), jnp.float32),
                   jax.ShapeDtypeStruct((B, D), jnp.float32)],
    )(text, visual, audio, W_t, b_t.reshape(1, D), W_v, b_v.reshape(1, D),
      W_a, b_a.reshape(1, D))

    n_tiles = (K + TILE_K - 1) // TILE_K
    topk_vals, topk_idx = pl.pallas_call(
        functools.partial(_sims_topk_kernel, n_bank=K, b_rows=B),
        grid=(n_tiles,),
        in_specs=[
            pl.BlockSpec((B, D), lambda j: (0, 0)),
            pl.BlockSpec((TILE_K, D), lambda j: (j, 0)),
        ],
        out_specs=[
            pl.BlockSpec((B, KNN), lambda j: (0, 0)),
            pl.BlockSpec((B, KNN), lambda j: (0, 0)),
        ],
        out_shape=[jax.ShapeDtypeStruct((B, KNN), jnp.float32),
                   jax.ShapeDtypeStruct((B, KNN), jnp.int32)],
    )(qn, bank)

    neighbors = _sc_gather(bank, topk_idx.reshape(B * KNN), B * KNN, D)
    neighbors = neighbors.reshape(B, KNN, D)

    proto_pad = jnp.zeros((8, D), jnp.float32).at[:C].set(prototypes)
    logits = pl.pallas_call(
        functools.partial(_finish_kernel, b_rows=B),
        out_shape=jax.ShapeDtypeStruct((B, 8), jnp.float32),
    )(fused, topk_vals, neighbors, proto_pad)

    return (logits[:, :C], topk_vals, topk_idx)
